# baseline (speedup 1.0000x reference)
import functools

import jax
import jax.numpy as jnp
from jax import lax
from jax.experimental import pallas as pl
from jax.experimental.pallas import tpu as pltpu, tpu_sc as plsc

_SQ3 = 0.5773502691896258

_NC, _NS, _L = 2, 16, 16
_NW = _NC * _NS
_NBUF = 5
_E = 10


@functools.lru_cache(maxsize=None)
def _tp_kernel(B, C):
    XW = 4 * C
    OW = 8 * C
    b_per_w = B // _NW
    E = _E
    assert B % _NW == 0 and b_per_w % (_NBUF * E) == 0
    n_chunks = b_per_w // E
    n_supers = n_chunks // _NBUF
    G = C // _L

    mesh = plsc.VectorSubcoreMesh(
        core_axis_name="c", subcore_axis_name="s",
        num_cores=_NC, num_subcores=_NS)

    @functools.partial(
        pl.kernel,
        out_type=jax.ShapeDtypeStruct((B * OW,), jnp.float32),
        mesh=mesh,
        scratch_types=(
            [pltpu.VMEM((E * XW,), jnp.float32) for _ in range(_NBUF)]
            + [pltpu.VMEM((E * XW,), jnp.float32) for _ in range(_NBUF)]
            + [pltpu.VMEM((E * OW,), jnp.float32) for _ in range(_NBUF)]
            + [pltpu.SemaphoreType.DMA for _ in range(3 * _NBUF)]
        ),
    )
    def k(x_hbm, y_hbm, o_hbm, *bufs):
        xvs = bufs[0:_NBUF]
        yvs = bufs[_NBUF:2 * _NBUF]
        ovs = bufs[2 * _NBUF:3 * _NBUF]
        sxs = bufs[3 * _NBUF:4 * _NBUF]
        sys_ = bufs[4 * _NBUF:5 * _NBUF]
        sos = bufs[5 * _NBUF:6 * _NBUF]

        wid = lax.axis_index("s") * _NC + lax.axis_index("c")
        base = wid * b_per_w

        def start_in(b, ci):
            e0 = (base + ci * E) * XW
            pltpu.make_async_copy(
                x_hbm.at[pl.ds(e0, E * XW)], xvs[b], sxs[b]).start()
            pltpu.make_async_copy(
                y_hbm.at[pl.ds(e0, E * XW)], yvs[b], sys_[b]).start()

        def wait_in(b):
            pltpu.make_async_copy(
                x_hbm.at[pl.ds(0, E * XW)], xvs[b], sxs[b]).wait()
            pltpu.make_async_copy(
                y_hbm.at[pl.ds(0, E * XW)], yvs[b], sys_[b]).wait()

        def start_out(b, ci):
            e0 = (base + ci * E) * OW
            pltpu.make_async_copy(
                ovs[b], o_hbm.at[pl.ds(e0, E * OW)], sos[b]).start()

        def wait_out(b):
            pltpu.make_async_copy(
                ovs[b], o_hbm.at[pl.ds(0, E * OW)], sos[b]).wait()

        def compute(b):
            xv, yv, ov = xvs[b], yvs[b], ovs[b]

            @plsc.parallel_loop(0, E, step=1, unroll=2)
            def edge_body(e):
                xb = e * XW
                ob = e * OW
                for g in range(G):
                    c0 = g * _L
                    x0 = xv[pl.ds(xb + 0 * C + c0, _L)]
                    x1 = xv[pl.ds(xb + 1 * C + c0, _L)]
                    x2 = xv[pl.ds(xb + 2 * C + c0, _L)]
                    x3 = xv[pl.ds(xb + 3 * C + c0, _L)]
                    y0 = yv[pl.ds(xb + 0 * C + c0, _L)]
                    y1 = yv[pl.ds(xb + 1 * C + c0, _L)]
                    y2 = yv[pl.ds(xb + 2 * C + c0, _L)]
                    y3 = yv[pl.ds(xb + 3 * C + c0, _L)]
                    ov[pl.ds(ob + 0 * C + c0, _L)] = x0 * y0
                    ov[pl.ds(ob + 1 * C + c0, _L)] = x0 * y1
                    ov[pl.ds(ob + 2 * C + c0, _L)] = x0 * y2
                    ov[pl.ds(ob + 3 * C + c0, _L)] = x0 * y3
                    ov[pl.ds(ob + 4 * C + c0, _L)] = x1 * y0
                    ov[pl.ds(ob + 5 * C + c0, _L)] = x2 * y0
                    ov[pl.ds(ob + 6 * C + c0, _L)] = x3 * y0
                    ov[pl.ds(ob + 7 * C + c0, _L)] = (
                        x1 * y1 + x2 * y2 + x3 * y3) * _SQ3

        for b in range(_NBUF):
            start_in(b, b)

        for b in range(_NBUF):
            wait_in(b)
            compute(b)
            start_out(b, b)
            start_in(b, b + _NBUF)

        def super_body(si, carry):
            for b in range(_NBUF):
                ci = si * _NBUF + b
                wait_in(b)
                wait_out(b)
                compute(b)
                start_out(b, ci)
                start_in(b, ci + _NBUF)
            return carry

        lax.fori_loop(1, n_supers - 1, super_body, 0)

        for b in range(_NBUF):
            wait_in(b)
            wait_out(b)
            compute(b)
            start_out(b, (n_supers - 1) * _NBUF + b)
        for b in range(_NBUF):
            wait_out(b)

    return k


def kernel(x, y):
    B, _, C = x.shape
    of = _tp_kernel(B, C)(x.reshape(-1), y.reshape(-1))
    return of.reshape(B, 8, C)
